# baseline (speedup 1.0000x reference)
import functools

import jax
import jax.numpy as jnp
from jax import lax
from jax.experimental import pallas as pl
from jax.experimental.pallas import tpu as pltpu
from jax.experimental.pallas import tpu_sc as plsc

_N = 10000
_NPAD = 12288
_E = 320000
_NW = 32
_EPW = _E // _NW
_K = 400
_CHUNKS = _EPW // _K
_RPT = _NPAD // 16
_ZB = 256

_BN = 2048
_GB = -(-_N // _BN)


def _pre_body(x_ref, ext_ref, wr_ref, wo_ref, ws_ref,
              z_ref, xd_ref, base_ref):
    ws = ws_ref[...]
    g = jnp.dot(wo_ref[144:272, :], ws, preferred_element_type=jnp.float32)
    wz = jnp.dot(wr_ref[0:128, :], g, preferred_element_type=jnp.float32)
    wxd = jnp.dot(wr_ref[128:256, :], g, preferred_element_type=jnp.float32)
    wbase = jnp.dot(wo_ref[0:128, :], ws, preferred_element_type=jnp.float32)
    wext = jnp.dot(wo_ref[128:144, :], ws, preferred_element_type=jnp.float32)
    xblk = x_ref[...]
    z_ref[...] = jnp.dot(xblk, wz, preferred_element_type=jnp.float32)
    xd_ref[...] = jnp.dot(xblk, wxd, preferred_element_type=jnp.float32)
    base_ref[...] = (
        jnp.dot(xblk, wbase, preferred_element_type=jnp.float32)
        + jnp.dot(ext_ref[...], wext, preferred_element_type=jnp.float32)
    )


def _fill(ref, rows, val):
    def body(i, carry):
        ref[i, :] = jnp.full((16,), val, jnp.float32)
        return carry
    lax.fori_loop(0, rows, body, 0)


def _zero_accs(zerov, accs, sid, sem):
    _fill(zerov, _ZB, 0.0)
    row0 = sid * _RPT
    zs = []
    for j in range(_RPT // _ZB):
        dst_slice = pl.ds(row0 + j * _ZB, _ZB)
        for acc in accs:
            zs.append(pltpu.async_copy(zerov, acc.at[dst_slice], sem))
    for h in zs:
        h.wait()
    plsc.subcore_barrier()


def _write_out(accs_outs, cid, sid):
    out_slice = pl.ds(cid * _NPAD + sid * _RPT, _RPT)
    acc_slice = pl.ds(sid * _RPT, _RPT)
    for acc, out in accs_outs:
        pltpu.sync_copy(acc.at[acc_slice], out.at[out_slice])


def _sc_a_body(z_hbm, src_hbm, dst_hbm, s1_out, dg_out,
               srcv0, dstv0, zrows0, srcv1, dstv1, zrows1,
               onesv, zerov, s1acc, dgacc,
               sem_ld0, sem_g0, sem_sc0, sem_ld1, sem_g1, sem_sc1):
    cid = lax.axis_index("c")
    sid = lax.axis_index("s")
    srcv = (srcv0, srcv1)
    dstv = (dstv0, dstv1)
    zrows = (zrows0, zrows1)
    sem_ld = (sem_ld0, sem_ld1)
    sem_g = (sem_g0, sem_g1)
    sem_sc = (sem_sc0, sem_sc1)

    _fill(onesv, _K, 1.0)
    _zero_accs(zerov, (s1acc, dgacc), sid, sem_ld0)

    off0 = (cid * 16 + sid) * _EPW

    def loads(c, s):
        base = off0 + c * _K
        return (
            pltpu.async_copy(src_hbm.at[pl.ds(base, _K)], srcv[s], sem_ld[s]),
            pltpu.async_copy(dst_hbm.at[pl.ds(base, _K)], dstv[s], sem_ld[s]),
        )

    ldp = [None, None]
    scp = [None, None]
    ldp[0] = loads(0, 0)
    for i in range(_CHUNKS):
        s = i & 1
        o = 1 - s
        for h in ldp[s]:
            h.wait()
        g = pltpu.async_copy(z_hbm.at[srcv[s]], zrows[s], sem_g[s])
        if scp[o] is not None:
            for h in scp[o]:
                h.wait()
        if i + 1 < _CHUNKS:
            ldp[o] = loads(i + 1, o)
        g.wait()
        scp[s] = (
            pltpu.async_copy(zrows[s], s1acc.at[dstv[s]], sem_sc[s], add=True),
            pltpu.async_copy(onesv, dgacc.at[dstv[s]], sem_sc[s], add=True),
        )
    for h in scp[(_CHUNKS - 1) & 1]:
        h.wait()
    plsc.subcore_barrier()
    _write_out(((s1acc, s1_out), (dgacc, dg_out)), cid, sid)


_EH = _E // 2
_EPWB = _EH // _NW
_KB = 1000
_CHUNKSB = _EPWB // _KB


def _sc_b_body(h0, dst_hbm, ea_hbm, s2_out,
               dstv0, earows0, dstv1, earows1, zerov, s2acc,
               sem_ld0, sem_sc0, sem_ld1, sem_sc1):
    cid = lax.axis_index("c")
    sid = lax.axis_index("s")
    dstv = (dstv0, dstv1)
    earows = (earows0, earows1)
    sem_ld = (sem_ld0, sem_ld1)
    sem_sc = (sem_sc0, sem_sc1)

    _zero_accs(zerov, (s2acc,), sid, sem_ld0)
    off0 = (cid * 16 + sid) * _EPWB

    def loads(c, s):
        base = off0 + c * _KB
        return (
            pltpu.async_copy(dst_hbm.at[pl.ds(h0 + base, _KB)], dstv[s],
                             sem_ld[s]),
            pltpu.async_copy(ea_hbm.at[pl.ds(base, _KB)], earows[s],
                             sem_ld[s]),
        )

    ldp = [None, None]
    scp = [None, None]
    ldp[0] = loads(0, 0)
    for i in range(_CHUNKSB):
        s = i & 1
        o = 1 - s
        for h in ldp[s]:
            h.wait()
        if scp[o] is not None:
            for h in scp[o]:
                h.wait()
        if i + 1 < _CHUNKSB:
            ldp[o] = loads(i + 1, o)
        scp[s] = (
            pltpu.async_copy(earows[s], s2acc.at[dstv[s]], sem_sc[s], add=True),
        )
    for h in scp[(_CHUNKSB - 1) & 1]:
        h.wait()
    plsc.subcore_barrier()
    _write_out(((s2acc, s2_out),), cid, sid)


_sc_mesh = plsc.VectorSubcoreMesh(core_axis_name="c", subcore_axis_name="s")
_sc_params = pltpu.CompilerParams(use_tc_tiling_on_sc=False)

_sc_segsum_a = functools.partial(
    pl.kernel,
    out_type=[jax.ShapeDtypeStruct((2 * _NPAD, 16), jnp.float32)] * 2,
    mesh=_sc_mesh,
    scratch_types=[
        pltpu.VMEM((_K,), jnp.int32),
        pltpu.VMEM((_K,), jnp.int32),
        pltpu.VMEM((_K, 16), jnp.float32),
        pltpu.VMEM((_K,), jnp.int32),
        pltpu.VMEM((_K,), jnp.int32),
        pltpu.VMEM((_K, 16), jnp.float32),
        pltpu.VMEM((_K, 16), jnp.float32),
        pltpu.VMEM((_ZB, 16), jnp.float32),
        pltpu.VMEM_SHARED((_NPAD, 16), jnp.float32),
        pltpu.VMEM_SHARED((_NPAD, 16), jnp.float32),
        pltpu.SemaphoreType.DMA,
        pltpu.SemaphoreType.DMA,
        pltpu.SemaphoreType.DMA,
        pltpu.SemaphoreType.DMA,
        pltpu.SemaphoreType.DMA,
        pltpu.SemaphoreType.DMA,
    ],
    compiler_params=_sc_params,
)(_sc_a_body)

def _make_sc_b(h0):
    return functools.partial(
        pl.kernel,
        out_type=[jax.ShapeDtypeStruct((2 * _NPAD, 16), jnp.float32)],
        mesh=_sc_mesh,
        scratch_types=[
            pltpu.VMEM((_KB,), jnp.int32),
            pltpu.VMEM((_KB, 16), jnp.float32),
            pltpu.VMEM((_KB,), jnp.int32),
            pltpu.VMEM((_KB, 16), jnp.float32),
            pltpu.VMEM((_ZB, 16), jnp.float32),
            pltpu.VMEM_SHARED((_NPAD, 16), jnp.float32),
            pltpu.SemaphoreType.DMA,
            pltpu.SemaphoreType.DMA,
            pltpu.SemaphoreType.DMA,
            pltpu.SemaphoreType.DMA,
        ],
        compiler_params=_sc_params,
    )(functools.partial(_sc_b_body, h0))


_sc_segsum_b1 = _make_sc_b(0)
_sc_segsum_b2 = _make_sc_b(_EH)


def _post_body(s1a_ref, s1b_ref, s2a1_ref, s2b1_ref, s2a2_ref, s2b2_ref,
               dga_ref, dgb_ref, xd_ref, base_ref,
               wr_ref, wo_ref, ws_ref, br_ref, bo_ref, bs_ref, out_ref):
    ws = ws_ref[...]
    g = jnp.dot(wo_ref[144:272, :], ws, preferred_element_type=jnp.float32)
    wea = jnp.dot(wr_ref[256:272, :], g, preferred_element_type=jnp.float32)
    c16 = jnp.dot(br_ref[...], g, preferred_element_type=jnp.float32)
    cb = jnp.dot(bo_ref[...], ws, preferred_element_type=jnp.float32) + bs_ref[...]

    s1 = s1a_ref[...] + s1b_ref[...]
    s2 = (s2a1_ref[...] + s2b1_ref[...]) + (s2a2_ref[...] + s2b2_ref[...])
    deg = dga_ref[...] + dgb_ref[...]
    scores = (
        base_ref[...] + s1
        + deg * (xd_ref[...] + c16)
        + jnp.dot(s2, wea, preferred_element_type=jnp.float32)
        + cb
    )
    m = jnp.max(scores, axis=1, keepdims=True)
    e = jnp.exp(scores - m)
    out_ref[...] = e / jnp.sum(e, axis=1, keepdims=True)


def kernel(x, edge_index, edge_attr, ext, W_r, b_r, W_o, b_o, W_s, b_s):
    n, ds = x.shape
    e = edge_index.shape[1]
    assert (n, ds, e) == (_N, 128, _E)

    grid = (_GB,)
    full = lambda shape: pl.BlockSpec(shape, lambda i: (0, 0))
    blk16 = pl.BlockSpec((_BN, 16), lambda i: (i, 0))

    z, xd16, base16 = pl.pallas_call(
        _pre_body,
        grid=grid,
        in_specs=[
            pl.BlockSpec((_BN, 128), lambda i: (i, 0)),
            blk16,
            full((272, 128)),
            full((272, 128)),
            full((128, 16)),
        ],
        out_specs=[blk16, blk16, blk16],
        out_shape=[jax.ShapeDtypeStruct((_N, 16), jnp.float32)] * 3,
    )(x, ext, W_r, W_o, W_s)

    src, dst = edge_index[0], edge_index[1]
    s1p, dgp = _sc_segsum_a(z, src, dst)
    (s2p1,) = _sc_segsum_b1(dst, edge_attr[:_EH])
    (s2p2,) = _sc_segsum_b2(dst, edge_attr[_EH:])

    hoff = _NPAD // _BN
    lof = pl.BlockSpec((_BN, 16), lambda i: (i, 0))
    hif = pl.BlockSpec((_BN, 16), lambda i: (i + hoff, 0))
    probs = pl.pallas_call(
        _post_body,
        grid=grid,
        in_specs=[
            lof, hif, lof, hif, lof, hif, lof, hif, blk16, blk16,
            full((272, 128)),
            full((272, 128)),
            full((128, 16)),
            full((1, 128)),
            full((1, 128)),
            full((1, 16)),
        ],
        out_specs=blk16,
        out_shape=jax.ShapeDtypeStruct((_N, 16), jnp.float32),
    )(s1p, s1p, s2p1, s2p1, s2p2, s2p2, dgp, dgp, xd16, base16, W_r, W_o, W_s,
      b_r.reshape(1, 128), b_o.reshape(1, 128), b_s.reshape(1, 16))
    return probs


# speedup vs baseline: 1.3643x; 1.3643x over previous
import functools

import jax
import jax.numpy as jnp
from jax import lax
from jax.experimental import pallas as pl
from jax.experimental.pallas import tpu as pltpu
from jax.experimental.pallas import tpu_sc as plsc

_N = 10000
_NPAD = 12288
_E = 320000
_NW = 32
_EPW = _E // _NW
_K = 400
_CHUNKS = _EPW // _K
_RPT = _NPAD // 16
_ZB = 256

_BN = 2048
_GB = -(-_N // _BN)


def _pre_body(x_ref, ext_ref, wr_ref, wo_ref, ws_ref,
              z_ref, xd_ref, base_ref):
    ws = ws_ref[...]
    g = jnp.dot(wo_ref[144:272, :], ws, preferred_element_type=jnp.float32)
    wz = jnp.dot(wr_ref[0:128, :], g, preferred_element_type=jnp.float32)
    wxd = jnp.dot(wr_ref[128:256, :], g, preferred_element_type=jnp.float32)
    wbase = jnp.dot(wo_ref[0:128, :], ws, preferred_element_type=jnp.float32)
    wext = jnp.dot(wo_ref[128:144, :], ws, preferred_element_type=jnp.float32)
    xblk = x_ref[...]
    z_ref[...] = jnp.dot(xblk, wz, preferred_element_type=jnp.float32)
    xd_ref[...] = jnp.dot(xblk, wxd, preferred_element_type=jnp.float32)
    base_ref[...] = (
        jnp.dot(xblk, wbase, preferred_element_type=jnp.float32)
        + jnp.dot(ext_ref[...], wext, preferred_element_type=jnp.float32)
    )


def _fill(ref, rows, val):
    def body(i, carry):
        ref[i, :] = jnp.full((16,), val, jnp.float32)
        return carry
    lax.fori_loop(0, rows, body, 0)


def _zero_accs(zerov, accs, sid, sem):
    _fill(zerov, _ZB, 0.0)
    row0 = sid * _RPT
    zs = []
    for j in range(_RPT // _ZB):
        dst_slice = pl.ds(row0 + j * _ZB, _ZB)
        for acc in accs:
            zs.append(pltpu.async_copy(zerov, acc.at[dst_slice], sem))
    for h in zs:
        h.wait()
    plsc.subcore_barrier()


def _write_out(accs_outs, cid, sid):
    out_slice = pl.ds(cid * _NPAD + sid * _RPT, _RPT)
    acc_slice = pl.ds(sid * _RPT, _RPT)
    for acc, out in accs_outs:
        pltpu.sync_copy(acc.at[acc_slice], out.at[out_slice])


def _sc_a_body(z_hbm, src_hbm, dst_hbm, s1_out, dg_out,
               srcv0, dstv0, zrows0, srcv1, dstv1, zrows1,
               onesv, zerov, s1acc, dgacc,
               sem_ld0, sem_g0, sem_sc0, sem_ld1, sem_g1, sem_sc1):
    cid = lax.axis_index("c")
    sid = lax.axis_index("s")
    srcv = (srcv0, srcv1)
    dstv = (dstv0, dstv1)
    zrows = (zrows0, zrows1)
    sem_ld = (sem_ld0, sem_ld1)
    sem_g = (sem_g0, sem_g1)
    sem_sc = (sem_sc0, sem_sc1)

    _fill(onesv, _K, 1.0)
    _zero_accs(zerov, (s1acc, dgacc), sid, sem_ld0)

    off0 = (cid * 16 + sid) * _EPW

    def loads(c, s):
        base = off0 + c * _K
        return (
            pltpu.async_copy(src_hbm.at[pl.ds(base, _K)], srcv[s], sem_ld[s]),
            pltpu.async_copy(dst_hbm.at[pl.ds(base, _K)], dstv[s], sem_ld[s]),
        )

    ldp = [None, None]
    scp = [None, None]
    ldp[0] = loads(0, 0)
    for i in range(_CHUNKS):
        s = i & 1
        o = 1 - s
        for h in ldp[s]:
            h.wait()
        g = pltpu.async_copy(z_hbm.at[srcv[s]], zrows[s], sem_g[s])
        if scp[o] is not None:
            for h in scp[o]:
                h.wait()
        if i + 1 < _CHUNKS:
            ldp[o] = loads(i + 1, o)
        g.wait()
        scp[s] = (
            pltpu.async_copy(zrows[s], s1acc.at[dstv[s]], sem_sc[s], add=True),
            pltpu.async_copy(onesv, dgacc.at[dstv[s]], sem_sc[s], add=True),
        )
    for h in scp[(_CHUNKS - 1) & 1]:
        h.wait()
    plsc.subcore_barrier()
    _write_out(((s1acc, s1_out), (dgacc, dg_out)), cid, sid)


def _sc_b_body(dst_hbm, ea_hbm, s2_out,
               dstv0, earows0, dstv1, earows1, zerov, s2acc,
               sem_ld0, sem_sc0, sem_ld1, sem_sc1):
    cid = lax.axis_index("c")
    sid = lax.axis_index("s")
    dstv = (dstv0, dstv1)
    earows = (earows0, earows1)
    sem_ld = (sem_ld0, sem_ld1)
    sem_sc = (sem_sc0, sem_sc1)

    _zero_accs(zerov, (s2acc,), sid, sem_ld0)
    off0 = (cid * 16 + sid) * _EPW

    def loads(c, s):
        base = off0 + c * _K
        return (
            pltpu.async_copy(dst_hbm.at[pl.ds(base, _K)], dstv[s], sem_ld[s]),
            pltpu.async_copy(ea_hbm.at[pl.ds(base, _K)], earows[s], sem_ld[s]),
        )

    ldp = [None, None]
    scp = [None, None]
    ldp[0] = loads(0, 0)
    for i in range(_CHUNKS):
        s = i & 1
        o = 1 - s
        for h in ldp[s]:
            h.wait()
        if scp[o] is not None:
            for h in scp[o]:
                h.wait()
        if i + 1 < _CHUNKS:
            ldp[o] = loads(i + 1, o)
        scp[s] = (
            pltpu.async_copy(earows[s], s2acc.at[dstv[s]], sem_sc[s], add=True),
        )
    for h in scp[(_CHUNKS - 1) & 1]:
        h.wait()
    plsc.subcore_barrier()
    _write_out(((s2acc, s2_out),), cid, sid)


_sc_mesh = plsc.VectorSubcoreMesh(core_axis_name="c", subcore_axis_name="s")
_sc_params = pltpu.CompilerParams(use_tc_tiling_on_sc=False)

_sc_segsum_a = functools.partial(
    pl.kernel,
    out_type=[jax.ShapeDtypeStruct((2 * _NPAD, 16), jnp.float32)] * 2,
    mesh=_sc_mesh,
    scratch_types=[
        pltpu.VMEM((_K,), jnp.int32),
        pltpu.VMEM((_K,), jnp.int32),
        pltpu.VMEM((_K, 16), jnp.float32),
        pltpu.VMEM((_K,), jnp.int32),
        pltpu.VMEM((_K,), jnp.int32),
        pltpu.VMEM((_K, 16), jnp.float32),
        pltpu.VMEM((_K, 16), jnp.float32),
        pltpu.VMEM((_ZB, 16), jnp.float32),
        pltpu.VMEM_SHARED((_NPAD, 16), jnp.float32),
        pltpu.VMEM_SHARED((_NPAD, 16), jnp.float32),
        pltpu.SemaphoreType.DMA,
        pltpu.SemaphoreType.DMA,
        pltpu.SemaphoreType.DMA,
        pltpu.SemaphoreType.DMA,
        pltpu.SemaphoreType.DMA,
        pltpu.SemaphoreType.DMA,
    ],
    compiler_params=_sc_params,
)(_sc_a_body)

_sc_segsum_b = functools.partial(
    pl.kernel,
    out_type=[jax.ShapeDtypeStruct((2 * _NPAD, 16), jnp.float32)],
    mesh=_sc_mesh,
    scratch_types=[
        pltpu.VMEM((_K,), jnp.int32),
        pltpu.VMEM((_K, 16), jnp.float32),
        pltpu.VMEM((_K,), jnp.int32),
        pltpu.VMEM((_K, 16), jnp.float32),
        pltpu.VMEM((_ZB, 16), jnp.float32),
        pltpu.VMEM_SHARED((_NPAD, 16), jnp.float32),
        pltpu.SemaphoreType.DMA,
        pltpu.SemaphoreType.DMA,
        pltpu.SemaphoreType.DMA,
        pltpu.SemaphoreType.DMA,
    ],
    compiler_params=_sc_params,
)(_sc_b_body)


def _post_body(s1a_ref, s1b_ref, s2a_ref, s2b_ref, dga_ref, dgb_ref,
               xd_ref, base_ref,
               wr_ref, wo_ref, ws_ref, br_ref, bo_ref, bs_ref, out_ref):
    ws = ws_ref[...]
    g = jnp.dot(wo_ref[144:272, :], ws, preferred_element_type=jnp.float32)
    wea = jnp.dot(wr_ref[256:272, :], g, preferred_element_type=jnp.float32)
    c16 = jnp.dot(br_ref[...], g, preferred_element_type=jnp.float32)
    cb = jnp.dot(bo_ref[...], ws, preferred_element_type=jnp.float32) + bs_ref[...]

    s1 = s1a_ref[...] + s1b_ref[...]
    s2 = s2a_ref[...] + s2b_ref[...]
    deg = dga_ref[...] + dgb_ref[...]
    scores = (
        base_ref[...] + s1
        + deg * (xd_ref[...] + c16)
        + jnp.dot(s2, wea, preferred_element_type=jnp.float32)
        + cb
    )
    m = jnp.max(scores, axis=1, keepdims=True)
    e = jnp.exp(scores - m)
    out_ref[...] = e / jnp.sum(e, axis=1, keepdims=True)


def kernel(x, edge_index, edge_attr, ext, W_r, b_r, W_o, b_o, W_s, b_s):
    n, ds = x.shape
    e = edge_index.shape[1]
    assert (n, ds, e) == (_N, 128, _E)

    grid = (_GB,)
    full = lambda shape: pl.BlockSpec(shape, lambda i: (0, 0))
    blk16 = pl.BlockSpec((_BN, 16), lambda i: (i, 0))

    z, xd16, base16 = pl.pallas_call(
        _pre_body,
        grid=grid,
        in_specs=[
            pl.BlockSpec((_BN, 128), lambda i: (i, 0)),
            blk16,
            full((272, 128)),
            full((272, 128)),
            full((128, 16)),
        ],
        out_specs=[blk16, blk16, blk16],
        out_shape=[jax.ShapeDtypeStruct((_N, 16), jnp.float32)] * 3,
    )(x, ext, W_r, W_o, W_s)

    src, dst = edge_index[0], edge_index[1]
    s1p, dgp = _sc_segsum_a(z, src, dst)
    (s2p,) = _sc_segsum_b(dst, edge_attr)

    s1f, s2f, dgf = s1p, s2p, dgp
    hoff = _NPAD // _BN
    lof = pl.BlockSpec((_BN, 16), lambda i: (i, 0))
    hif = pl.BlockSpec((_BN, 16), lambda i: (i + hoff, 0))
    probs = pl.pallas_call(
        _post_body,
        grid=grid,
        in_specs=[
            lof, hif, lof, hif, lof, hif, blk16, blk16,
            full((272, 128)),
            full((272, 128)),
            full((128, 16)),
            full((1, 128)),
            full((1, 128)),
            full((1, 16)),
        ],
        out_specs=blk16,
        out_shape=jax.ShapeDtypeStruct((_N, 16), jnp.float32),
    )(s1f, s1f, s2f, s2f, dgf, dgf, xd16, base16, W_r, W_o, W_s,
      b_r.reshape(1, 128), b_o.reshape(1, 128), b_s.reshape(1, 16))
    return probs
